# baseline (speedup 1.0000x reference)
import functools
import math

import jax
import jax.numpy as jnp
import numpy as np
from jax import lax
from jax.experimental import pallas as pl
from jax.experimental.pallas import tpu as pltpu

HIDDEN = 4096
N_HEADS = 32
N_KV = 8
HD = 128
SEQ = 2048
DQ = N_HEADS * HD
DKV = N_KV * HD
ROPE_THETA = 500000.0
SCALE = 1.0 / math.sqrt(HD)

_INV_FREQ2 = np.tile(
    1.0 / (ROPE_THETA ** (np.arange(0, HD, 2, dtype=np.float32) / HD)),
    2).reshape(1, HD)
_SIGN = np.concatenate([-np.ones(HD // 2, np.float32),
                        np.ones(HD // 2, np.float32)]).reshape(1, HD)


def _hadamard_matrix(n):
    H = np.array([[1.0]], dtype=np.float32)
    while H.shape[0] < n:
        H = np.block([[H, H], [H, -H]]).astype(np.float32)
    return H

_HM = _hadamard_matrix(HD)

_DN_T = (((1,), (1,)), ((), ()))
_DN_N = (((1,), (0,)), ((), ()))


def _tables_kernel(x_ref, if_ref, sg_ref, xb_ref, cos_ref, ssin_ref, *, bs):
    i = pl.program_id(0)
    xb_ref[...] = x_ref[...].astype(jnp.bfloat16)
    pos = (i * bs + lax.broadcasted_iota(jnp.int32, (bs, 1), 0)
           ).astype(jnp.float32)
    f = pos * if_ref[...]
    cos_ref[...] = jnp.cos(f)
    ssin_ref[...] = sg_ref[...] * jnp.sin(f)


def _rope_head(x, cos, ssin):
    return x * cos + pltpu.roll(x, HD // 2, 1) * ssin


def _qproj_kernel(x_ref, w_ref, cos_ref, ssin_ref, q_ref, acc_ref, *, nk, nc):
    ki = pl.program_id(1)

    def dot():
        return lax.dot_general(
            x_ref[...], w_ref[...].astype(jnp.bfloat16), _DN_T,
            preferred_element_type=jnp.float32)

    @pl.when(ki == 0)
    def _():
        acc_ref[...] = dot()

    @pl.when(ki > 0)
    def _():
        acc_ref[...] += dot()

    @pl.when(ki == nk - 1)
    def _():
        acc = acc_ref[...]
        cos = cos_ref[...]
        ssin = ssin_ref[...]
        for h in range(nc // HD):
            b = h * HD
            q_ref[:, b:b + HD] = _rope_head(
                acc[:, b:b + HD], cos, ssin).astype(jnp.bfloat16)


def _matmul_kernel(a_ref, w_ref, o_ref, acc_ref, *, nk):
    ki = pl.program_id(1)

    def dot():
        return lax.dot_general(
            a_ref[...], w_ref[...].astype(jnp.bfloat16), _DN_T,
            preferred_element_type=jnp.float32)

    @pl.when(ki == 0)
    def _():
        acc_ref[...] = dot()

    @pl.when(ki > 0)
    def _():
        acc_ref[...] += dot()

    @pl.when(ki == nk - 1)
    def _():
        o_ref[...] = acc_ref[...].astype(o_ref.dtype)


def _hadamard_kernel(k_ref, hm_ref, h_ref):
    hm = hm_ref[...]
    for h in range(N_KV):
        b = h * HD
        h_ref[:, b:b + HD] = lax.dot_general(
            k_ref[:, b:b + HD], hm, _DN_N, preferred_element_type=jnp.float32)


def _attn_kernel(q_ref, k_ref, v_ref, o_ref, *, bq, bk, groups):
    qi = pl.program_id(1)
    mrows = groups * bq
    q = jnp.concatenate(
        [q_ref[:, h * HD:(h + 1) * HD] for h in range(groups)], axis=0)
    q = (q.astype(jnp.float32) * (SCALE * math.log2(math.e))
         ).astype(jnp.bfloat16)
    neg = jnp.finfo(jnp.float32).min

    def step(j, carry, masked):
        m, l, acc = carry
        kb = k_ref[pl.ds(j * bk, bk), :]
        vb = v_ref[pl.ds(j * bk, bk), :]
        s = lax.dot_general(q, kb, _DN_T, preferred_element_type=jnp.float32)
        if masked:
            rowm = lax.broadcasted_iota(jnp.int32, (mrows, bk), 0) & (bq - 1)
            col = lax.broadcasted_iota(jnp.int32, (mrows, bk), 1)
            s = jnp.where(col <= rowm, s, neg)
        m2 = jnp.maximum(m, jnp.max(s, axis=1, keepdims=True))
        alpha = jnp.exp2(m - m2)
        p = jnp.exp2(s - m2)
        l2 = l * alpha + jnp.sum(p, axis=1, keepdims=True)
        acc2 = acc * alpha + lax.dot_general(
            p.astype(jnp.bfloat16), vb, _DN_N,
            preferred_element_type=jnp.float32)
        return m2, l2, acc2

    m0 = jnp.full((mrows, 1), neg, jnp.float32)
    l0 = jnp.zeros((mrows, 1), jnp.float32)
    a0 = jnp.zeros((mrows, HD), jnp.float32)
    nfull = qi * (bq // bk)
    carry = lax.fori_loop(0, nfull, lambda j, c: step(j, c, False),
                          (m0, l0, a0))
    m, l, acc = step(nfull, carry, True)
    o = acc / l
    for h in range(groups):
        o_ref[:, h * HD:(h + 1) * HD] = (
            o[h * bq:(h + 1) * bq, :].astype(jnp.bfloat16))


def _tables_call(x, interpret=False):
    BS = 512
    invf = jnp.asarray(_INV_FREQ2)
    sign = jnp.asarray(_SIGN)
    return pl.pallas_call(
        functools.partial(_tables_kernel, bs=BS),
        grid=(SEQ // BS,),
        in_specs=[
            pl.BlockSpec((BS, HIDDEN), lambda i: (i, 0)),
            pl.BlockSpec((1, HD), lambda i: (0, 0)),
            pl.BlockSpec((1, HD), lambda i: (0, 0)),
        ],
        out_specs=(
            pl.BlockSpec((BS, HIDDEN), lambda i: (i, 0)),
            pl.BlockSpec((BS, HD), lambda i: (i, 0)),
            pl.BlockSpec((BS, HD), lambda i: (i, 0)),
        ),
        out_shape=(jax.ShapeDtypeStruct((SEQ, HIDDEN), jnp.bfloat16),
                   jax.ShapeDtypeStruct((SEQ, HD), jnp.float32),
                   jax.ShapeDtypeStruct((SEQ, HD), jnp.float32)),
        compiler_params=pltpu.CompilerParams(
            dimension_semantics=("arbitrary",)),
        interpret=interpret,
    )(x, invf, sign)


def _qproj_call(xb, W, cos, ssin, interpret=False):
    NC, KC = 1024, 1024
    dout = W.shape[0]
    nn, nk = dout // NC, HIDDEN // KC
    return pl.pallas_call(
        functools.partial(_qproj_kernel, nk=nk, nc=NC),
        grid=(nn, nk),
        in_specs=[
            pl.BlockSpec((SEQ, KC), lambda ni, ki: (0, ki)),
            pl.BlockSpec((NC, KC), lambda ni, ki: (ni, ki)),
            pl.BlockSpec((SEQ, HD), lambda ni, ki: (0, 0)),
            pl.BlockSpec((SEQ, HD), lambda ni, ki: (0, 0)),
        ],
        out_specs=pl.BlockSpec((SEQ, NC), lambda ni, ki: (0, ni)),
        out_shape=jax.ShapeDtypeStruct((SEQ, dout), jnp.bfloat16),
        scratch_shapes=[pltpu.VMEM((SEQ, NC), jnp.float32)],
        compiler_params=pltpu.CompilerParams(
            dimension_semantics=("arbitrary", "arbitrary")),
        interpret=interpret,
    )(xb, W, cos, ssin)


def _matmul_call(a, W, out_dtype, interpret=False):
    NC, KC = 1024, 1024
    dout, din = W.shape
    nn, nk = dout // NC, din // KC
    return pl.pallas_call(
        functools.partial(_matmul_kernel, nk=nk),
        grid=(nn, nk),
        in_specs=[
            pl.BlockSpec((SEQ, KC), lambda ni, ki: (0, ki)),
            pl.BlockSpec((NC, KC), lambda ni, ki: (ni, ki)),
        ],
        out_specs=pl.BlockSpec((SEQ, NC), lambda ni, ki: (0, ni)),
        out_shape=jax.ShapeDtypeStruct((SEQ, dout), out_dtype),
        scratch_shapes=[pltpu.VMEM((SEQ, NC), jnp.float32)],
        compiler_params=pltpu.CompilerParams(
            dimension_semantics=("arbitrary", "arbitrary")),
        interpret=interpret,
    )(a, W)


def _hadamard_call(k, interpret=False):
    BS = 512
    hm = jnp.asarray(_HM)
    return pl.pallas_call(
        _hadamard_kernel,
        grid=(SEQ // BS,),
        in_specs=[
            pl.BlockSpec((BS, DKV), lambda i: (i, 0)),
            pl.BlockSpec((HD, HD), lambda i: (0, 0)),
        ],
        out_specs=pl.BlockSpec((BS, DKV), lambda i: (i, 0)),
        out_shape=jax.ShapeDtypeStruct((SEQ, DKV), jnp.float32),
        compiler_params=pltpu.CompilerParams(
            dimension_semantics=("arbitrary",)),
        interpret=interpret,
    )(k, hm)


def _attn_call(q, k, v, interpret=False):
    BQ = BK = 512
    nq = SEQ // BQ
    groups = N_HEADS // N_KV
    GD = groups * HD
    return pl.pallas_call(
        functools.partial(_attn_kernel, bq=BQ, bk=BK, groups=groups),
        grid=(N_KV, nq),
        in_specs=[
            pl.BlockSpec((BQ, GD), lambda g, qi: (qi, g)),
            pl.BlockSpec((SEQ, HD), lambda g, qi: (0, g)),
            pl.BlockSpec((SEQ, HD), lambda g, qi: (0, g)),
        ],
        out_specs=pl.BlockSpec((BQ, GD), lambda g, qi: (qi, g)),
        out_shape=jax.ShapeDtypeStruct((SEQ, DQ), jnp.bfloat16),
        compiler_params=pltpu.CompilerParams(
            dimension_semantics=("arbitrary", "arbitrary")),
        interpret=interpret,
    )(q, k, v)


def kernel(hidden_states, position_ids, Wq, Wk, Wv, Wo, interpret=False):
    xb, cos, ssin = _tables_call(hidden_states[0], interpret=interpret)
    q = _qproj_call(xb, Wq, cos, ssin, interpret=interpret)
    k = _qproj_call(xb, Wk, cos, ssin, interpret=interpret)
    v = _matmul_call(xb, Wv, jnp.bfloat16, interpret=interpret)
    had = _hadamard_call(k, interpret=interpret)
    attn = _attn_call(q, k, v, interpret=interpret)
    out = _matmul_call(attn, Wo, jnp.float32, interpret=interpret)
    return out[None], had.reshape(SEQ, N_KV, HD)


# speedup vs baseline: 1.0176x; 1.0176x over previous
import functools
import math

import jax
import jax.numpy as jnp
import numpy as np
from jax import lax
from jax.experimental import pallas as pl
from jax.experimental.pallas import tpu as pltpu

HIDDEN = 4096
N_HEADS = 32
N_KV = 8
HD = 128
SEQ = 2048
DQ = N_HEADS * HD
DKV = N_KV * HD
ROPE_THETA = 500000.0
SCALE = 1.0 / math.sqrt(HD)

_INV_FREQ2 = np.tile(
    1.0 / (ROPE_THETA ** (np.arange(0, HD, 2, dtype=np.float32) / HD)),
    2).reshape(1, HD)
_SIGN = np.concatenate([-np.ones(HD // 2, np.float32),
                        np.ones(HD // 2, np.float32)]).reshape(1, HD)


def _hadamard_matrix(n):
    H = np.array([[1.0]], dtype=np.float32)
    while H.shape[0] < n:
        H = np.block([[H, H], [H, -H]]).astype(np.float32)
    return H

_HM = _hadamard_matrix(HD)

_DN_T = (((1,), (1,)), ((), ()))
_DN_N = (((1,), (0,)), ((), ()))


def _tables_kernel(x_ref, if_ref, sg_ref, xb_ref, cos_ref, ssin_ref, *, bs):
    i = pl.program_id(0)
    xb_ref[...] = x_ref[...].astype(jnp.bfloat16)
    pos = (i * bs + lax.broadcasted_iota(jnp.int32, (bs, 1), 0)
           ).astype(jnp.float32)
    f = pos * if_ref[...]
    cos_ref[...] = jnp.cos(f)
    ssin_ref[...] = sg_ref[...] * jnp.sin(f)


def _rope_head(x, cos, ssin):
    return x * cos + pltpu.roll(x, HD // 2, 1) * ssin


def _qproj_kernel(x_ref, w_ref, cos_ref, ssin_ref, q_ref, acc_ref, *, nk, nc):
    ki = pl.program_id(1)

    def dot():
        return lax.dot_general(
            x_ref[...], w_ref[...].astype(jnp.bfloat16), _DN_T,
            preferred_element_type=jnp.float32)

    @pl.when(ki == 0)
    def _():
        acc_ref[...] = dot()

    @pl.when(ki > 0)
    def _():
        acc_ref[...] += dot()

    @pl.when(ki == nk - 1)
    def _():
        acc = acc_ref[...]
        cos = cos_ref[...]
        ssin = ssin_ref[...]
        for h in range(nc // HD):
            b = h * HD
            q_ref[:, b:b + HD] = _rope_head(
                acc[:, b:b + HD], cos, ssin).astype(jnp.bfloat16)


def _kvproj_kernel(x_ref, wk_ref, wv_ref, cos_ref, ssin_ref,
                   k_ref, v_ref, acck_ref, accv_ref, *, nk):
    ki = pl.program_id(0)

    def dotk():
        return lax.dot_general(
            x_ref[...], wk_ref[...].astype(jnp.bfloat16), _DN_T,
            preferred_element_type=jnp.float32)

    def dotv():
        return lax.dot_general(
            x_ref[...], wv_ref[...].astype(jnp.bfloat16), _DN_T,
            preferred_element_type=jnp.float32)

    @pl.when(ki == 0)
    def _():
        acck_ref[...] = dotk()
        accv_ref[...] = dotv()

    @pl.when(ki > 0)
    def _():
        acck_ref[...] += dotk()
        accv_ref[...] += dotv()

    @pl.when(ki == nk - 1)
    def _():
        acck = acck_ref[...]
        cos = cos_ref[...]
        ssin = ssin_ref[...]
        for h in range(N_KV):
            b = h * HD
            k_ref[:, b:b + HD] = _rope_head(
                acck[:, b:b + HD], cos, ssin).astype(jnp.bfloat16)
        v_ref[...] = accv_ref[...].astype(jnp.bfloat16)


def _kvproj_call(xb, Wk, Wv, cos, ssin):
    KC = 512
    nk = HIDDEN // KC
    return pl.pallas_call(
        functools.partial(_kvproj_kernel, nk=nk),
        grid=(nk,),
        in_specs=[
            pl.BlockSpec((SEQ, KC), lambda ki: (0, ki)),
            pl.BlockSpec((DKV, KC), lambda ki: (0, ki)),
            pl.BlockSpec((DKV, KC), lambda ki: (0, ki)),
            pl.BlockSpec((SEQ, HD), lambda ki: (0, 0)),
            pl.BlockSpec((SEQ, HD), lambda ki: (0, 0)),
        ],
        out_specs=(
            pl.BlockSpec((SEQ, DKV), lambda ki: (0, 0)),
            pl.BlockSpec((SEQ, DKV), lambda ki: (0, 0)),
        ),
        out_shape=(jax.ShapeDtypeStruct((SEQ, DKV), jnp.bfloat16),
                   jax.ShapeDtypeStruct((SEQ, DKV), jnp.bfloat16)),
        scratch_shapes=[pltpu.VMEM((SEQ, DKV), jnp.float32),
                        pltpu.VMEM((SEQ, DKV), jnp.float32)],
        compiler_params=pltpu.CompilerParams(
            dimension_semantics=("arbitrary",)),
    )(xb, Wk, Wv, cos, ssin)


def _matmul_kernel(a_ref, w_ref, o_ref, acc_ref, *, nk):
    ki = pl.program_id(1)

    def dot():
        return lax.dot_general(
            a_ref[...], w_ref[...].astype(jnp.bfloat16), _DN_T,
            preferred_element_type=jnp.float32)

    @pl.when(ki == 0)
    def _():
        acc_ref[...] = dot()

    @pl.when(ki > 0)
    def _():
        acc_ref[...] += dot()

    @pl.when(ki == nk - 1)
    def _():
        o_ref[...] = acc_ref[...].astype(o_ref.dtype)


def _hadamard_kernel(k_ref, hm_ref, h_ref):
    hm = hm_ref[...]
    for h in range(N_KV):
        b = h * HD
        h_ref[:, b:b + HD] = lax.dot_general(
            k_ref[:, b:b + HD], hm, _DN_N, preferred_element_type=jnp.float32)


def _attn_kernel(q_ref, k_ref, v_ref, o_ref, *, bq, bk, groups):
    qi = pl.program_id(1)
    mrows = groups * bq
    q = jnp.concatenate(
        [q_ref[:, h * HD:(h + 1) * HD] for h in range(groups)], axis=0)
    q = (q.astype(jnp.float32) * (SCALE * math.log2(math.e))
         ).astype(jnp.bfloat16)
    neg = jnp.finfo(jnp.float32).min

    def step(j, carry, masked):
        m, l, acc = carry
        kb = k_ref[pl.ds(j * bk, bk), :]
        vb = v_ref[pl.ds(j * bk, bk), :]
        s = lax.dot_general(q, kb, _DN_T, preferred_element_type=jnp.float32)
        if masked:
            rowm = lax.broadcasted_iota(jnp.int32, (mrows, bk), 0) & (bq - 1)
            col = lax.broadcasted_iota(jnp.int32, (mrows, bk), 1)
            s = jnp.where(col <= rowm, s, neg)
        m2 = jnp.maximum(m, jnp.max(s, axis=1, keepdims=True))
        alpha = jnp.exp2(m - m2)
        p = jnp.exp2(s - m2)
        l2 = l * alpha + jnp.sum(p, axis=1, keepdims=True)
        acc2 = acc * alpha + lax.dot_general(
            p.astype(jnp.bfloat16), vb, _DN_N,
            preferred_element_type=jnp.float32)
        return m2, l2, acc2

    m0 = jnp.full((mrows, 1), neg, jnp.float32)
    l0 = jnp.zeros((mrows, 1), jnp.float32)
    a0 = jnp.zeros((mrows, HD), jnp.float32)
    nfull = qi * (bq // bk)
    carry = lax.fori_loop(0, nfull, lambda j, c: step(j, c, False),
                          (m0, l0, a0))
    m, l, acc = step(nfull, carry, True)
    o = acc / l
    for h in range(groups):
        o_ref[:, h * HD:(h + 1) * HD] = (
            o[h * bq:(h + 1) * bq, :].astype(jnp.bfloat16))


def _tables_call(x):
    BS = 512
    invf = jnp.asarray(_INV_FREQ2)
    sign = jnp.asarray(_SIGN)
    return pl.pallas_call(
        functools.partial(_tables_kernel, bs=BS),
        grid=(SEQ // BS,),
        in_specs=[
            pl.BlockSpec((BS, HIDDEN), lambda i: (i, 0)),
            pl.BlockSpec((1, HD), lambda i: (0, 0)),
            pl.BlockSpec((1, HD), lambda i: (0, 0)),
        ],
        out_specs=(
            pl.BlockSpec((BS, HIDDEN), lambda i: (i, 0)),
            pl.BlockSpec((BS, HD), lambda i: (i, 0)),
            pl.BlockSpec((BS, HD), lambda i: (i, 0)),
        ),
        out_shape=(jax.ShapeDtypeStruct((SEQ, HIDDEN), jnp.bfloat16),
                   jax.ShapeDtypeStruct((SEQ, HD), jnp.float32),
                   jax.ShapeDtypeStruct((SEQ, HD), jnp.float32)),
        compiler_params=pltpu.CompilerParams(
            dimension_semantics=("arbitrary",)),
    )(x, invf, sign)


def _qproj_call(xb, W, cos, ssin):
    NC, KC = 1024, 1024
    dout = W.shape[0]
    nn, nk = dout // NC, HIDDEN // KC
    return pl.pallas_call(
        functools.partial(_qproj_kernel, nk=nk, nc=NC),
        grid=(nn, nk),
        in_specs=[
            pl.BlockSpec((SEQ, KC), lambda ni, ki: (0, ki)),
            pl.BlockSpec((NC, KC), lambda ni, ki: (ni, ki)),
            pl.BlockSpec((SEQ, HD), lambda ni, ki: (0, 0)),
            pl.BlockSpec((SEQ, HD), lambda ni, ki: (0, 0)),
        ],
        out_specs=pl.BlockSpec((SEQ, NC), lambda ni, ki: (0, ni)),
        out_shape=jax.ShapeDtypeStruct((SEQ, dout), jnp.bfloat16),
        scratch_shapes=[pltpu.VMEM((SEQ, NC), jnp.float32)],
        compiler_params=pltpu.CompilerParams(
            dimension_semantics=("arbitrary", "arbitrary")),
    )(xb, W, cos, ssin)


def _matmul_call(a, W, out_dtype):
    NC, KC = 1024, 1024
    dout, din = W.shape
    nn, nk = dout // NC, din // KC
    return pl.pallas_call(
        functools.partial(_matmul_kernel, nk=nk),
        grid=(nn, nk),
        in_specs=[
            pl.BlockSpec((SEQ, KC), lambda ni, ki: (0, ki)),
            pl.BlockSpec((NC, KC), lambda ni, ki: (ni, ki)),
        ],
        out_specs=pl.BlockSpec((SEQ, NC), lambda ni, ki: (0, ni)),
        out_shape=jax.ShapeDtypeStruct((SEQ, dout), out_dtype),
        scratch_shapes=[pltpu.VMEM((SEQ, NC), jnp.float32)],
        compiler_params=pltpu.CompilerParams(
            dimension_semantics=("arbitrary", "arbitrary")),
    )(a, W)


def _hadamard_call(k):
    BS = 512
    hm = jnp.asarray(_HM)
    return pl.pallas_call(
        _hadamard_kernel,
        grid=(SEQ // BS,),
        in_specs=[
            pl.BlockSpec((BS, DKV), lambda i: (i, 0)),
            pl.BlockSpec((HD, HD), lambda i: (0, 0)),
        ],
        out_specs=pl.BlockSpec((BS, DKV), lambda i: (i, 0)),
        out_shape=jax.ShapeDtypeStruct((SEQ, DKV), jnp.float32),
        compiler_params=pltpu.CompilerParams(
            dimension_semantics=("arbitrary",)),
    )(k, hm)


def _attn_call(q, k, v):
    BQ = BK = 512
    nq = SEQ // BQ
    groups = N_HEADS // N_KV
    GD = groups * HD
    return pl.pallas_call(
        functools.partial(_attn_kernel, bq=BQ, bk=BK, groups=groups),
        grid=(N_KV, nq),
        in_specs=[
            pl.BlockSpec((BQ, GD), lambda g, qi: (qi, g)),
            pl.BlockSpec((SEQ, HD), lambda g, qi: (0, g)),
            pl.BlockSpec((SEQ, HD), lambda g, qi: (0, g)),
        ],
        out_specs=pl.BlockSpec((BQ, GD), lambda g, qi: (qi, g)),
        out_shape=jax.ShapeDtypeStruct((SEQ, DQ), jnp.bfloat16),
        compiler_params=pltpu.CompilerParams(
            dimension_semantics=("arbitrary", "arbitrary")),
    )(q, k, v)


def kernel(hidden_states, position_ids, Wq, Wk, Wv, Wo):
    xb, cos, ssin = _tables_call(hidden_states[0])
    q = _qproj_call(xb, Wq, cos, ssin)
    k, v = _kvproj_call(xb, Wk, Wv, cos, ssin)
    had = _hadamard_call(k)
    attn = _attn_call(q, k, v)
    out = _matmul_call(attn, Wo, jnp.float32)
    return out[None], had.reshape(SEQ, N_KV, HD)


# speedup vs baseline: 1.0195x; 1.0019x over previous
import functools
import math

import jax
import jax.numpy as jnp
import numpy as np
from jax import lax
from jax.experimental import pallas as pl
from jax.experimental.pallas import tpu as pltpu

HIDDEN = 4096
N_HEADS = 32
N_KV = 8
HD = 128
SEQ = 2048
DQ = N_HEADS * HD
DKV = N_KV * HD
GROUPS = N_HEADS // N_KV
ROPE_THETA = 500000.0
SCALE = 1.0 / math.sqrt(HD)

_INV_FREQ2 = np.tile(
    1.0 / (ROPE_THETA ** (np.arange(0, HD, 2, dtype=np.float32) / HD)),
    2).reshape(1, HD)
_SIGN = np.concatenate([-np.ones(HD // 2, np.float32),
                        np.ones(HD // 2, np.float32)]).reshape(1, HD)


def _hadamard_matrix(n):
    H = np.array([[1.0]], dtype=np.float32)
    while H.shape[0] < n:
        H = np.block([[H, H], [H, -H]]).astype(np.float32)
    return H

_HM = _hadamard_matrix(HD)

_DN_T = (((1,), (1,)), ((), ()))
_DN_N = (((1,), (0,)), ((), ()))


def _tables_kernel(x_ref, if_ref, sg_ref, xb_ref, cos_ref, ssin_ref, *, bs):
    i = pl.program_id(0)
    xb_ref[...] = x_ref[...].astype(jnp.bfloat16)
    pos = (i * bs + lax.broadcasted_iota(jnp.int32, (bs, 1), 0)
           ).astype(jnp.float32)
    f = pos * if_ref[...]
    cos_ref[...] = jnp.cos(f)
    ssin_ref[...] = sg_ref[...] * jnp.sin(f)


def _rope_head(x, cos, ssin):
    return x * cos + pltpu.roll(x, HD // 2, 1) * ssin


def _qproj_kernel(x_ref, w_ref, cos_ref, ssin_ref, q_ref, acc_ref, *, nk, nc):
    ki = pl.program_id(1)

    def dot():
        return lax.dot_general(
            x_ref[...], w_ref[...].astype(jnp.bfloat16), _DN_T,
            preferred_element_type=jnp.float32)

    @pl.when(ki == 0)
    def _():
        acc_ref[...] = dot()

    @pl.when(ki > 0)
    def _():
        acc_ref[...] += dot()

    @pl.when(ki == nk - 1)
    def _():
        acc = acc_ref[...]
        cos = cos_ref[...]
        ssin = ssin_ref[...]
        for h in range(nc // HD):
            b = h * HD
            q_ref[h // GROUPS, h % GROUPS, :, :] = _rope_head(
                acc[:, b:b + HD], cos, ssin).astype(jnp.bfloat16)


def _kvproj_kernel(x_ref, wk_ref, wv_ref, cos_ref, ssin_ref,
                   k_ref, v_ref, acck_ref, accv_ref, *, nk):
    ki = pl.program_id(0)

    def dotk():
        return lax.dot_general(
            x_ref[...], wk_ref[...].astype(jnp.bfloat16), _DN_T,
            preferred_element_type=jnp.float32)

    def dotv():
        return lax.dot_general(
            x_ref[...], wv_ref[...].astype(jnp.bfloat16), _DN_T,
            preferred_element_type=jnp.float32)

    @pl.when(ki == 0)
    def _():
        acck_ref[...] = dotk()
        accv_ref[...] = dotv()

    @pl.when(ki > 0)
    def _():
        acck_ref[...] += dotk()
        accv_ref[...] += dotv()

    @pl.when(ki == nk - 1)
    def _():
        acck = acck_ref[...]
        cos = cos_ref[...]
        ssin = ssin_ref[...]
        for h in range(N_KV):
            b = h * HD
            k_ref[:, b:b + HD] = _rope_head(
                acck[:, b:b + HD], cos, ssin).astype(jnp.bfloat16)
        v_ref[...] = accv_ref[...].astype(jnp.bfloat16)


def _kvproj_call(xb, Wk, Wv, cos, ssin):
    KC = 512
    nk = HIDDEN // KC
    return pl.pallas_call(
        functools.partial(_kvproj_kernel, nk=nk),
        grid=(nk,),
        in_specs=[
            pl.BlockSpec((SEQ, KC), lambda ki: (0, ki)),
            pl.BlockSpec((DKV, KC), lambda ki: (0, ki)),
            pl.BlockSpec((DKV, KC), lambda ki: (0, ki)),
            pl.BlockSpec((SEQ, HD), lambda ki: (0, 0)),
            pl.BlockSpec((SEQ, HD), lambda ki: (0, 0)),
        ],
        out_specs=(
            pl.BlockSpec((SEQ, DKV), lambda ki: (0, 0)),
            pl.BlockSpec((SEQ, DKV), lambda ki: (0, 0)),
        ),
        out_shape=(jax.ShapeDtypeStruct((SEQ, DKV), jnp.bfloat16),
                   jax.ShapeDtypeStruct((SEQ, DKV), jnp.bfloat16)),
        scratch_shapes=[pltpu.VMEM((SEQ, DKV), jnp.float32),
                        pltpu.VMEM((SEQ, DKV), jnp.float32)],
        compiler_params=pltpu.CompilerParams(
            dimension_semantics=("arbitrary",)),
    )(xb, Wk, Wv, cos, ssin)


def _matmul_kernel(a_ref, w_ref, o_ref, acc_ref, *, nk):
    ki = pl.program_id(1)

    def dot():
        return lax.dot_general(
            a_ref[...], w_ref[...].astype(jnp.bfloat16), _DN_T,
            preferred_element_type=jnp.float32)

    @pl.when(ki == 0)
    def _():
        acc_ref[...] = dot()

    @pl.when(ki > 0)
    def _():
        acc_ref[...] += dot()

    @pl.when(ki == nk - 1)
    def _():
        o_ref[...] = acc_ref[...].astype(o_ref.dtype)


def _hadamard_kernel(k_ref, hm_ref, h_ref):
    hm = hm_ref[...]
    for h in range(N_KV):
        b = h * HD
        h_ref[:, b:b + HD] = lax.dot_general(
            k_ref[:, b:b + HD], hm, _DN_N, preferred_element_type=jnp.float32)


def _attn_kernel(q_ref, k_ref, v_ref, o_ref, *, bq, bk, groups):
    qi = pl.program_id(1)
    mrows = groups * bq
    q = jnp.reshape(q_ref[...], (mrows, HD))
    q = (q.astype(jnp.float32) * (SCALE * math.log2(math.e))
         ).astype(jnp.bfloat16)
    neg = jnp.finfo(jnp.float32).min

    def step(j, carry, masked):
        m, l, acc = carry
        kb = k_ref[pl.ds(j * bk, bk), :]
        vb = v_ref[pl.ds(j * bk, bk), :]
        s = lax.dot_general(q, kb, _DN_T, preferred_element_type=jnp.float32)
        if masked:
            rowm = lax.broadcasted_iota(jnp.int32, (mrows, bk), 0) & (bq - 1)
            col = lax.broadcasted_iota(jnp.int32, (mrows, bk), 1)
            s = jnp.where(col <= rowm, s, neg)
        m2 = jnp.maximum(m, jnp.max(s, axis=1, keepdims=True))
        alpha = jnp.exp2(m - m2)
        p = jnp.exp2(s - m2)
        l2 = l * alpha + jnp.sum(p, axis=1, keepdims=True)
        acc2 = acc * alpha + lax.dot_general(
            p.astype(jnp.bfloat16), vb, _DN_N,
            preferred_element_type=jnp.float32)
        return m2, l2, acc2

    m0 = jnp.full((mrows, 1), neg, jnp.float32)
    l0 = jnp.zeros((mrows, 1), jnp.float32)
    a0 = jnp.zeros((mrows, HD), jnp.float32)
    nfull = qi * (bq // bk)
    carry = lax.fori_loop(0, nfull, lambda j, c: step(j, c, False),
                          (m0, l0, a0))
    m, l, acc = step(nfull, carry, True)
    o = acc / l
    for h in range(groups):
        o_ref[:, h * HD:(h + 1) * HD] = (
            o[h * bq:(h + 1) * bq, :].astype(jnp.bfloat16))


def _tables_call(x):
    BS = 512
    invf = jnp.asarray(_INV_FREQ2)
    sign = jnp.asarray(_SIGN)
    return pl.pallas_call(
        functools.partial(_tables_kernel, bs=BS),
        grid=(SEQ // BS,),
        in_specs=[
            pl.BlockSpec((BS, HIDDEN), lambda i: (i, 0)),
            pl.BlockSpec((1, HD), lambda i: (0, 0)),
            pl.BlockSpec((1, HD), lambda i: (0, 0)),
        ],
        out_specs=(
            pl.BlockSpec((BS, HIDDEN), lambda i: (i, 0)),
            pl.BlockSpec((BS, HD), lambda i: (i, 0)),
            pl.BlockSpec((BS, HD), lambda i: (i, 0)),
        ),
        out_shape=(jax.ShapeDtypeStruct((SEQ, HIDDEN), jnp.bfloat16),
                   jax.ShapeDtypeStruct((SEQ, HD), jnp.float32),
                   jax.ShapeDtypeStruct((SEQ, HD), jnp.float32)),
        compiler_params=pltpu.CompilerParams(
            dimension_semantics=("arbitrary",)),
    )(x, invf, sign)


def _qproj_call(xb, W, cos, ssin):
    NC, KC = 1024, 1024
    gpb = NC // HD // GROUPS
    nn, nk = DQ // NC, HIDDEN // KC
    return pl.pallas_call(
        functools.partial(_qproj_kernel, nk=nk, nc=NC),
        grid=(nn, nk),
        in_specs=[
            pl.BlockSpec((SEQ, KC), lambda ni, ki: (0, ki)),
            pl.BlockSpec((NC, KC), lambda ni, ki: (ni, ki)),
            pl.BlockSpec((SEQ, HD), lambda ni, ki: (0, 0)),
            pl.BlockSpec((SEQ, HD), lambda ni, ki: (0, 0)),
        ],
        out_specs=pl.BlockSpec((gpb, GROUPS, SEQ, HD),
                               lambda ni, ki: (ni, 0, 0, 0)),
        out_shape=jax.ShapeDtypeStruct((N_KV, GROUPS, SEQ, HD), jnp.bfloat16),
        scratch_shapes=[pltpu.VMEM((SEQ, NC), jnp.float32)],
        compiler_params=pltpu.CompilerParams(
            dimension_semantics=("arbitrary", "arbitrary")),
    )(xb, W, cos, ssin)


def _matmul_call(a, W, out_dtype):
    NC, KC = 1024, 1024
    dout, din = W.shape
    nn, nk = dout // NC, din // KC
    return pl.pallas_call(
        functools.partial(_matmul_kernel, nk=nk),
        grid=(nn, nk),
        in_specs=[
            pl.BlockSpec((SEQ, KC), lambda ni, ki: (0, ki)),
            pl.BlockSpec((NC, KC), lambda ni, ki: (ni, ki)),
        ],
        out_specs=pl.BlockSpec((SEQ, NC), lambda ni, ki: (0, ni)),
        out_shape=jax.ShapeDtypeStruct((SEQ, dout), out_dtype),
        scratch_shapes=[pltpu.VMEM((SEQ, NC), jnp.float32)],
        compiler_params=pltpu.CompilerParams(
            dimension_semantics=("arbitrary", "arbitrary")),
    )(a, W)


def _hadamard_call(k):
    BS = 512
    hm = jnp.asarray(_HM)
    return pl.pallas_call(
        _hadamard_kernel,
        grid=(SEQ // BS,),
        in_specs=[
            pl.BlockSpec((BS, DKV), lambda i: (i, 0)),
            pl.BlockSpec((HD, HD), lambda i: (0, 0)),
        ],
        out_specs=pl.BlockSpec((BS, DKV), lambda i: (i, 0)),
        out_shape=jax.ShapeDtypeStruct((SEQ, DKV), jnp.float32),
        compiler_params=pltpu.CompilerParams(
            dimension_semantics=("arbitrary",)),
    )(k, hm)


def _attn_call(q, k, v):
    BQ = BK = 512
    nq = SEQ // BQ
    groups = N_HEADS // N_KV
    GD = groups * HD
    return pl.pallas_call(
        functools.partial(_attn_kernel, bq=BQ, bk=BK, groups=groups),
        grid=(N_KV, nq),
        in_specs=[
            pl.BlockSpec((1, groups, BQ, HD), lambda g, qi: (g, 0, qi, 0)),
            pl.BlockSpec((SEQ, HD), lambda g, qi: (0, g)),
            pl.BlockSpec((SEQ, HD), lambda g, qi: (0, g)),
        ],
        out_specs=pl.BlockSpec((BQ, GD), lambda g, qi: (qi, g)),
        out_shape=jax.ShapeDtypeStruct((SEQ, DQ), jnp.bfloat16),
        compiler_params=pltpu.CompilerParams(
            dimension_semantics=("arbitrary", "arbitrary")),
    )(q, k, v)


def kernel(hidden_states, position_ids, Wq, Wk, Wv, Wo):
    xb, cos, ssin = _tables_call(hidden_states[0])
    q = _qproj_call(xb, Wq, cos, ssin)
    k, v = _kvproj_call(xb, Wk, Wv, cos, ssin)
    had = _hadamard_call(k)
    attn = _attn_call(q, k, v)
    out = _matmul_call(attn, Wo, jnp.float32)
    return out[None], had.reshape(SEQ, N_KV, HD)


# speedup vs baseline: 1.0268x; 1.0072x over previous
import functools
import math

import jax
import jax.numpy as jnp
import numpy as np
from jax import lax
from jax.experimental import pallas as pl
from jax.experimental.pallas import tpu as pltpu

HIDDEN = 4096
N_HEADS = 32
N_KV = 8
HD = 128
SEQ = 2048
DQ = N_HEADS * HD
DKV = N_KV * HD
GROUPS = N_HEADS // N_KV
ROPE_THETA = 500000.0
SCALE = 1.0 / math.sqrt(HD)

_INV_FREQ2 = np.tile(
    1.0 / (ROPE_THETA ** (np.arange(0, HD, 2, dtype=np.float32) / HD)),
    2).reshape(1, HD)
_SIGN = np.concatenate([-np.ones(HD // 2, np.float32),
                        np.ones(HD // 2, np.float32)]).reshape(1, HD)


def _hadamard_matrix(n):
    H = np.array([[1.0]], dtype=np.float32)
    while H.shape[0] < n:
        H = np.block([[H, H], [H, -H]]).astype(np.float32)
    return H

_HM = _hadamard_matrix(HD)

_DN_T = (((1,), (1,)), ((), ()))
_DN_N = (((1,), (0,)), ((), ()))


def _tables_kernel(x_ref, if_ref, sg_ref,
                   xb_ref, cos_ref, ssin_ref, cosq_ref, ssinq_ref, *, bs):
    i = pl.program_id(0)
    xb_ref[...] = x_ref[...].astype(jnp.bfloat16)
    pos = (i * bs + lax.broadcasted_iota(jnp.int32, (bs, 1), 0)
           ).astype(jnp.float32)
    f = pos * if_ref[...]
    cos = jnp.cos(f)
    ssin = sg_ref[...] * jnp.sin(f)
    cos_ref[...] = cos
    ssin_ref[...] = ssin
    qs = SCALE * math.log2(math.e)
    cosq_ref[...] = cos * qs
    ssinq_ref[...] = ssin * qs


def _rope_head(x, cos, ssin):
    return x * cos + pltpu.roll(x, HD // 2, 1) * ssin


def _qproj_kernel(x_ref, w_ref, cos_ref, ssin_ref, q_ref, acc_ref, *, nk, nc):
    ki = pl.program_id(1)

    def dot():
        return lax.dot_general(
            x_ref[...], w_ref[...].astype(jnp.bfloat16), _DN_T,
            preferred_element_type=jnp.float32)

    @pl.when(ki == 0)
    def _():
        acc_ref[...] = dot()

    @pl.when(ki > 0)
    def _():
        acc_ref[...] += dot()

    @pl.when(ki == nk - 1)
    def _():
        acc = acc_ref[...]
        cos = cos_ref[...]
        ssin = ssin_ref[...]
        for h in range(nc // HD):
            b = h * HD
            q_ref[h // GROUPS, h % GROUPS, :, :] = _rope_head(
                acc[:, b:b + HD], cos, ssin).astype(jnp.bfloat16)


def _kvproj_kernel(x_ref, wk_ref, wv_ref, cos_ref, ssin_ref,
                   k_ref, v_ref, acck_ref, accv_ref, *, nk):
    ki = pl.program_id(0)

    def dotk():
        return lax.dot_general(
            x_ref[...], wk_ref[...].astype(jnp.bfloat16), _DN_T,
            preferred_element_type=jnp.float32)

    def dotv():
        return lax.dot_general(
            x_ref[...], wv_ref[...].astype(jnp.bfloat16), _DN_T,
            preferred_element_type=jnp.float32)

    @pl.when(ki == 0)
    def _():
        acck_ref[...] = dotk()
        accv_ref[...] = dotv()

    @pl.when(ki > 0)
    def _():
        acck_ref[...] += dotk()
        accv_ref[...] += dotv()

    @pl.when(ki == nk - 1)
    def _():
        acck = acck_ref[...]
        cos = cos_ref[...]
        ssin = ssin_ref[...]
        for h in range(N_KV):
            b = h * HD
            k_ref[:, b:b + HD] = _rope_head(
                acck[:, b:b + HD], cos, ssin).astype(jnp.bfloat16)
        v_ref[...] = accv_ref[...].astype(jnp.bfloat16)


def _kvproj_call(xb, Wk, Wv, cos, ssin):
    KC = 512
    nk = HIDDEN // KC
    return pl.pallas_call(
        functools.partial(_kvproj_kernel, nk=nk),
        grid=(nk,),
        in_specs=[
            pl.BlockSpec((SEQ, KC), lambda ki: (0, ki)),
            pl.BlockSpec((DKV, KC), lambda ki: (0, ki)),
            pl.BlockSpec((DKV, KC), lambda ki: (0, ki)),
            pl.BlockSpec((SEQ, HD), lambda ki: (0, 0)),
            pl.BlockSpec((SEQ, HD), lambda ki: (0, 0)),
        ],
        out_specs=(
            pl.BlockSpec((SEQ, DKV), lambda ki: (0, 0)),
            pl.BlockSpec((SEQ, DKV), lambda ki: (0, 0)),
        ),
        out_shape=(jax.ShapeDtypeStruct((SEQ, DKV), jnp.bfloat16),
                   jax.ShapeDtypeStruct((SEQ, DKV), jnp.bfloat16)),
        scratch_shapes=[pltpu.VMEM((SEQ, DKV), jnp.float32),
                        pltpu.VMEM((SEQ, DKV), jnp.float32)],
        compiler_params=pltpu.CompilerParams(
            dimension_semantics=("arbitrary",)),
    )(xb, Wk, Wv, cos, ssin)


def _matmul_kernel(a_ref, w_ref, o_ref, acc_ref, *, nk):
    ki = pl.program_id(1)

    def dot():
        return lax.dot_general(
            a_ref[...], w_ref[...].astype(jnp.bfloat16), _DN_T,
            preferred_element_type=jnp.float32)

    @pl.when(ki == 0)
    def _():
        acc_ref[...] = dot()

    @pl.when(ki > 0)
    def _():
        acc_ref[...] += dot()

    @pl.when(ki == nk - 1)
    def _():
        o_ref[...] = acc_ref[...].astype(o_ref.dtype)


def _hadamard_kernel(k_ref, hm_ref, h_ref):
    hm = hm_ref[...]
    for h in range(N_KV):
        b = h * HD
        h_ref[:, b:b + HD] = lax.dot_general(
            k_ref[:, b:b + HD], hm, _DN_N, preferred_element_type=jnp.float32)


def _attn_kernel(q_ref, k_ref, v_ref, o_ref, *, bq, bk, groups):
    qi = pl.program_id(1)
    mrows = groups * bq
    q = jnp.reshape(q_ref[...], (mrows, HD))
    neg = jnp.finfo(jnp.float32).min

    def step(j, carry, masked):
        m, l, acc = carry
        kb = k_ref[pl.ds(j * bk, bk), :]
        vb = v_ref[pl.ds(j * bk, bk), :]
        s = lax.dot_general(q, kb, _DN_T, preferred_element_type=jnp.float32)
        if masked:
            rowm = lax.broadcasted_iota(jnp.int32, (mrows, bk), 0) & (bq - 1)
            col = lax.broadcasted_iota(jnp.int32, (mrows, bk), 1)
            s = jnp.where(col <= rowm, s, neg)
        m2 = jnp.maximum(m, jnp.max(s, axis=1, keepdims=True))
        alpha = jnp.exp2(m - m2)
        p = jnp.exp2(s - m2)
        l2 = l * alpha + jnp.sum(p, axis=1, keepdims=True)
        acc2 = acc * alpha + lax.dot_general(
            p.astype(jnp.bfloat16), vb, _DN_N,
            preferred_element_type=jnp.float32)
        return m2, l2, acc2

    m0 = jnp.full((mrows, 1), neg, jnp.float32)
    l0 = jnp.zeros((mrows, 1), jnp.float32)
    a0 = jnp.zeros((mrows, HD), jnp.float32)
    nfull = qi * (bq // bk)
    carry = lax.fori_loop(0, nfull, lambda j, c: step(j, c, False),
                          (m0, l0, a0))
    m, l, acc = step(nfull, carry, True)
    o = acc / l
    for h in range(groups):
        o_ref[:, h * HD:(h + 1) * HD] = (
            o[h * bq:(h + 1) * bq, :].astype(jnp.bfloat16))


def _tables_call(x):
    BS = 512
    invf = jnp.asarray(_INV_FREQ2)
    sign = jnp.asarray(_SIGN)
    return pl.pallas_call(
        functools.partial(_tables_kernel, bs=BS),
        grid=(SEQ // BS,),
        in_specs=[
            pl.BlockSpec((BS, HIDDEN), lambda i: (i, 0)),
            pl.BlockSpec((1, HD), lambda i: (0, 0)),
            pl.BlockSpec((1, HD), lambda i: (0, 0)),
        ],
        out_specs=(
            pl.BlockSpec((BS, HIDDEN), lambda i: (i, 0)),
            pl.BlockSpec((BS, HD), lambda i: (i, 0)),
            pl.BlockSpec((BS, HD), lambda i: (i, 0)),
            pl.BlockSpec((BS, HD), lambda i: (i, 0)),
            pl.BlockSpec((BS, HD), lambda i: (i, 0)),
        ),
        out_shape=(jax.ShapeDtypeStruct((SEQ, HIDDEN), jnp.bfloat16),
                   jax.ShapeDtypeStruct((SEQ, HD), jnp.float32),
                   jax.ShapeDtypeStruct((SEQ, HD), jnp.float32),
                   jax.ShapeDtypeStruct((SEQ, HD), jnp.float32),
                   jax.ShapeDtypeStruct((SEQ, HD), jnp.float32)),
        compiler_params=pltpu.CompilerParams(
            dimension_semantics=("arbitrary",)),
    )(x, invf, sign)


def _qproj_call(xb, W, cos, ssin):
    NC, KC = 1024, 1024
    gpb = NC // HD // GROUPS
    nn, nk = DQ // NC, HIDDEN // KC
    return pl.pallas_call(
        functools.partial(_qproj_kernel, nk=nk, nc=NC),
        grid=(nn, nk),
        in_specs=[
            pl.BlockSpec((SEQ, KC), lambda ni, ki: (0, ki)),
            pl.BlockSpec((NC, KC), lambda ni, ki: (ni, ki)),
            pl.BlockSpec((SEQ, HD), lambda ni, ki: (0, 0)),
            pl.BlockSpec((SEQ, HD), lambda ni, ki: (0, 0)),
        ],
        out_specs=pl.BlockSpec((gpb, GROUPS, SEQ, HD),
                               lambda ni, ki: (ni, 0, 0, 0)),
        out_shape=jax.ShapeDtypeStruct((N_KV, GROUPS, SEQ, HD), jnp.bfloat16),
        scratch_shapes=[pltpu.VMEM((SEQ, NC), jnp.float32)],
        compiler_params=pltpu.CompilerParams(
            dimension_semantics=("arbitrary", "arbitrary")),
    )(xb, W, cos, ssin)


def _matmul_call(a, W, out_dtype):
    NC, KC = 1024, 1024
    dout, din = W.shape
    nn, nk = dout // NC, din // KC
    return pl.pallas_call(
        functools.partial(_matmul_kernel, nk=nk),
        grid=(nn, nk),
        in_specs=[
            pl.BlockSpec((SEQ, KC), lambda ni, ki: (0, ki)),
            pl.BlockSpec((NC, KC), lambda ni, ki: (ni, ki)),
        ],
        out_specs=pl.BlockSpec((SEQ, NC), lambda ni, ki: (0, ni)),
        out_shape=jax.ShapeDtypeStruct((SEQ, dout), out_dtype),
        scratch_shapes=[pltpu.VMEM((SEQ, NC), jnp.float32)],
        compiler_params=pltpu.CompilerParams(
            dimension_semantics=("arbitrary", "arbitrary")),
    )(a, W)


def _hadamard_call(k):
    BS = 512
    hm = jnp.asarray(_HM)
    return pl.pallas_call(
        _hadamard_kernel,
        grid=(SEQ // BS,),
        in_specs=[
            pl.BlockSpec((BS, DKV), lambda i: (i, 0)),
            pl.BlockSpec((HD, HD), lambda i: (0, 0)),
        ],
        out_specs=pl.BlockSpec((BS, DKV), lambda i: (i, 0)),
        out_shape=jax.ShapeDtypeStruct((SEQ, DKV), jnp.float32),
        compiler_params=pltpu.CompilerParams(
            dimension_semantics=("arbitrary",)),
    )(k, hm)


def _attn_call(q, k, v):
    BQ = BK = 512
    nq = SEQ // BQ
    groups = N_HEADS // N_KV
    GD = groups * HD
    return pl.pallas_call(
        functools.partial(_attn_kernel, bq=BQ, bk=BK, groups=groups),
        grid=(N_KV, nq),
        in_specs=[
            pl.BlockSpec((1, groups, BQ, HD), lambda g, qi: (g, 0, qi, 0)),
            pl.BlockSpec((SEQ, HD), lambda g, qi: (0, g)),
            pl.BlockSpec((SEQ, HD), lambda g, qi: (0, g)),
        ],
        out_specs=pl.BlockSpec((BQ, GD), lambda g, qi: (qi, g)),
        out_shape=jax.ShapeDtypeStruct((SEQ, DQ), jnp.bfloat16),
        compiler_params=pltpu.CompilerParams(
            dimension_semantics=("arbitrary", "arbitrary")),
    )(q, k, v)


def kernel(hidden_states, position_ids, Wq, Wk, Wv, Wo):
    xb, cos, ssin, cosq, ssinq = _tables_call(hidden_states[0])
    q = _qproj_call(xb, Wq, cosq, ssinq)
    k, v = _kvproj_call(xb, Wk, Wv, cos, ssin)
    had = _hadamard_call(k)
    attn = _attn_call(q, k, v)
    out = _matmul_call(attn, Wo, jnp.float32)
    return out[None], had.reshape(SEQ, N_KV, HD)
